# Initial kernel scaffold; baseline (speedup 1.0000x reference)
import jax
import jax.numpy as jnp
from jax.experimental import pallas as pl


def kernel(x, ln1_w, Wq, bq, Wak, Wbk, Wav, Wbv, Wo, bo, ln2_w, Wr, br, Wn, bn, Wsw, Wdown):
    raise NotImplementedError("write your pallas kernel here")


# speedup vs baseline: 5.9328x; 5.9328x over previous
import jax
import jax.numpy as jnp
from jax.experimental import pallas as pl
from jax.experimental.pallas import tpu as pltpu

B, T, D = 1, 2048, 768
H, HS, LAT = 12, 64, 32
E, TOPK = 64, 1
NFF = 1536
HID = int(2 * NFF / 3)
CAP = int(B * T * TOPK / E * 1.0)


def _attn_kernel(x_ref, w1_ref, wqe_ref, wqo_ref, bqe_ref, bqo_ref,
                 wak_ref, wbke_ref, wbko_ref, wav_ref, wbv_ref,
                 cos_ref, sin_ref, o_ref, h_scr):
    h_idx = pl.program_id(0)

    @pl.when(h_idx == 0)
    def _():
        xv = x_ref[...]
        ms = jnp.mean(xv * xv, axis=-1, keepdims=True)
        h_scr[...] = xv * jax.lax.rsqrt(ms + 1e-5) * w1_ref[...]

    h = h_scr[...]
    cos = cos_ref[...]
    sin = sin_ref[...]

    qe = jnp.dot(h, wqe_ref[0], preferred_element_type=jnp.float32) + bqe_ref[0]
    qo = jnp.dot(h, wqo_ref[0], preferred_element_type=jnp.float32) + bqo_ref[0]
    ka = jnp.dot(h, wak_ref[0], preferred_element_type=jnp.float32)
    ke = jnp.dot(ka, wbke_ref[0], preferred_element_type=jnp.float32)
    ko = jnp.dot(ka, wbko_ref[0], preferred_element_type=jnp.float32)
    va = jnp.dot(h, wav_ref[0], preferred_element_type=jnp.float32)
    v = jnp.dot(va, wbv_ref[0], preferred_element_type=jnp.float32)

    qr = jnp.concatenate([qe * cos - qo * sin, qe * sin + qo * cos], axis=-1)
    kr = jnp.concatenate([ke * cos - ko * sin, ke * sin + ko * cos], axis=-1)

    scores = jax.lax.dot_general(qr, kr, (((1,), (1,)), ((), ())),
                                 preferred_element_type=jnp.float32)
    scores = scores / 8.0
    m = jnp.max(scores, axis=-1, keepdims=True)
    p = jnp.exp(scores - m)
    s = jnp.sum(p, axis=-1, keepdims=True)
    attn = p / s
    o_ref[0] = jnp.dot(attn, v, preferred_element_type=jnp.float32)


def _router_kernel(x_ref, o_ref, wo_ref, bo_ref, w2_ref, wr_ref, br_ref,
                   wn_ref, bn_ref, eps_ref,
                   x2_ref, h2_ref, vrank_ref, aux_ref):
    x2 = x_ref[...] + bo_ref[...]
    for hh in range(H):
        x2 = x2 + jnp.dot(o_ref[hh], wo_ref[hh],
                          preferred_element_type=jnp.float32)
    x2_ref[...] = x2
    ms = jnp.mean(x2 * x2, axis=-1, keepdims=True)
    h2 = x2 * jax.lax.rsqrt(ms + 1e-5) * w2_ref[...]
    h2_ref[...] = h2

    logits = jnp.dot(h2, wr_ref[...], preferred_element_type=jnp.float32) + br_ref[...]
    npre = jnp.dot(h2, wn_ref[...], preferred_element_type=jnp.float32) + bn_ref[...]
    nscale = jnp.logaddexp(npre, 0.0)
    noisy = logits + eps_ref[...] * nscale

    rowmax = jnp.max(noisy, axis=-1, keepdims=True)
    lane = jax.lax.broadcasted_iota(jnp.int32, (T, E), 1)
    first = jnp.min(jnp.where(noisy == rowmax, lane, E), axis=-1, keepdims=True)
    oh = (lane == first).astype(jnp.float32)

    counts = jnp.sum(oh, axis=0, keepdims=True)
    aux_ref[...] = jnp.sum((counts / T - 1.0 / E) ** 2, keepdims=True)[:, :1]

    r = jax.lax.broadcasted_iota(jnp.int32, (T, T), 0)
    c = jax.lax.broadcasted_iota(jnp.int32, (T, T), 1)
    tri = (r >= c).astype(jnp.bfloat16)
    rank = jax.lax.dot_general(tri, oh.astype(jnp.bfloat16),
                               (((1,), (0,)), ((), ())),
                               preferred_element_type=jnp.float32) - 1.0
    valid = (oh > 0.0) & (rank < CAP)
    vrank = jnp.where(valid, rank, -1.0)
    vrank_ref[...] = jnp.transpose(vrank)


def _expert_kernel(h2_ref, x2_ref, vrank_ref, wsw_ref, wdown_ref, out_ref):
    e = pl.program_id(0)
    vr = vrank_ref[0].astype(jnp.int32)
    slot = jax.lax.broadcasted_iota(jnp.int32, (CAP, T), 0)
    p_sel = (slot == vr).astype(jnp.float32)

    xin = jnp.dot(p_sel, h2_ref[...], preferred_element_type=jnp.float32)
    pp = jnp.dot(xin, wsw_ref[0], preferred_element_type=jnp.float32)
    p1 = pp[:, :HID]
    p2 = pp[:, HID:]
    g = (p1 * jax.lax.logistic(p1)) * p2
    eo = jnp.dot(g, wdown_ref[0], preferred_element_type=jnp.float32)
    contrib = jax.lax.dot_general(p_sel, eo, (((0,), (0,)), ((), ())),
                                  preferred_element_type=jnp.float32)

    @pl.when(e == 0)
    def _():
        out_ref[...] = x2_ref[...] + contrib

    @pl.when(e != 0)
    def _():
        out_ref[...] = out_ref[...] + contrib


def kernel(x, ln1_w, Wq, bq, Wak, Wbk, Wav, Wbv, Wo, bo, ln2_w, Wr, br, Wn,
           bn, Wsw, Wdown):
    x2d = x.reshape(T, D)

    inv_freq = 1.0 / (10000.0 ** (jnp.arange(0, HS, 2, dtype=jnp.float32) / HS))
    pos = jnp.arange(T, dtype=jnp.float32)
    ang = pos[:, None] * inv_freq[None, :]
    cos = jnp.cos(ang)
    sin = jnp.sin(ang)

    wqe, wqo = Wq[:, :, 0::2], Wq[:, :, 1::2]
    bqe = bq[:, 0::2].reshape(H, 1, LAT)
    bqo = bq[:, 1::2].reshape(H, 1, LAT)
    wbke, wbko = Wbk[:, :, 0::2], Wbk[:, :, 1::2]

    full = lambda *_: tuple(0 for _ in range(2))
    per_head3 = pl.BlockSpec((1, D, LAT), lambda h: (h, 0, 0))

    o = pl.pallas_call(
        _attn_kernel,
        grid=(H,),
        in_specs=[
            pl.BlockSpec((T, D), lambda h: (0, 0)),
            pl.BlockSpec((1, D), lambda h: (0, 0)),
            per_head3,
            per_head3,
            pl.BlockSpec((1, 1, LAT), lambda h: (h, 0, 0)),
            pl.BlockSpec((1, 1, LAT), lambda h: (h, 0, 0)),
            per_head3,
            pl.BlockSpec((1, LAT, LAT), lambda h: (h, 0, 0)),
            pl.BlockSpec((1, LAT, LAT), lambda h: (h, 0, 0)),
            per_head3,
            pl.BlockSpec((1, LAT, HS), lambda h: (h, 0, 0)),
            pl.BlockSpec((T, LAT), lambda h: (0, 0)),
            pl.BlockSpec((T, LAT), lambda h: (0, 0)),
        ],
        out_specs=pl.BlockSpec((1, T, HS), lambda h: (h, 0, 0)),
        out_shape=jax.ShapeDtypeStruct((H, T, HS), jnp.float32),
        scratch_shapes=[pltpu.VMEM((T, D), jnp.float32)],
    )(x2d, ln1_w.reshape(1, D), wqe, wqo, bqe, bqo, Wak, wbke, wbko, Wav,
      Wbv, cos, sin)

    eps = jax.random.normal(jax.random.key(1234), (B, T, E),
                            jnp.float32).reshape(T, E)

    x2, h2, vrank_t, aux = pl.pallas_call(
        _router_kernel,
        in_specs=[
            pl.BlockSpec((T, D), full),
            pl.BlockSpec((H, T, HS), lambda: (0, 0, 0)),
            pl.BlockSpec((H, HS, D), lambda: (0, 0, 0)),
            pl.BlockSpec((1, D), full),
            pl.BlockSpec((1, D), full),
            pl.BlockSpec((D, E), full),
            pl.BlockSpec((1, E), full),
            pl.BlockSpec((D, E), full),
            pl.BlockSpec((1, E), full),
            pl.BlockSpec((T, E), full),
        ],
        out_specs=[
            pl.BlockSpec((T, D), full),
            pl.BlockSpec((T, D), full),
            pl.BlockSpec((E, T), full),
            pl.BlockSpec((1, 1), full),
        ],
        out_shape=[
            jax.ShapeDtypeStruct((T, D), jnp.float32),
            jax.ShapeDtypeStruct((T, D), jnp.float32),
            jax.ShapeDtypeStruct((E, T), jnp.float32),
            jax.ShapeDtypeStruct((1, 1), jnp.float32),
        ],
    )(x2d, o, Wo.reshape(H, HS, D), bo.reshape(1, D), ln2_w.reshape(1, D), Wr,
      br.reshape(1, E), Wn, bn.reshape(1, E), eps)

    vrank3 = vrank_t.reshape(E, 1, T)

    out2d = pl.pallas_call(
        _expert_kernel,
        grid=(E,),
        in_specs=[
            pl.BlockSpec((T, D), lambda e: (0, 0)),
            pl.BlockSpec((T, D), lambda e: (0, 0)),
            pl.BlockSpec((1, 1, T), lambda e: (e, 0, 0)),
            pl.BlockSpec((1, D, 2 * HID), lambda e: (e, 0, 0)),
            pl.BlockSpec((1, HID, D), lambda e: (e, 0, 0)),
        ],
        out_specs=pl.BlockSpec((T, D), lambda e: (0, 0)),
        out_shape=jax.ShapeDtypeStruct((T, D), jnp.float32),
    )(h2, x2, vrank3, Wsw, Wdown)

    return out2d.reshape(B, T, D), aux.reshape(())


# speedup vs baseline: 6.1831x; 1.0422x over previous
import jax
import jax.numpy as jnp
from jax.experimental import pallas as pl
from jax.experimental.pallas import tpu as pltpu

B, T, D = 1, 2048, 768
H, HS, LAT = 12, 64, 32
E, TOPK = 64, 1
NFF = 1536
HID = int(2 * NFF / 3)
CAP = int(B * T * TOPK / E * 1.0)

HL = H * LAT
QC = 4 * HL


def _proj_kernel(x_ref, w1_ref, wcat_ref, bcat_ref, bdke_ref, bdko_ref,
                 bdv_ref, cos_ref, sin_ref, pk_ref):
    xv = x_ref[...]
    ms = jnp.mean(xv * xv, axis=-1, keepdims=True)
    hv = xv * jax.lax.rsqrt(ms + 1e-5) * w1_ref[...]
    qav = jnp.dot(hv, wcat_ref[...],
                  preferred_element_type=jnp.float32) + bcat_ref[...]
    qe = qav[:, 0 * HL:1 * HL]
    qo = qav[:, 1 * HL:2 * HL]
    ka = qav[:, 2 * HL:3 * HL]
    va = qav[:, 3 * HL:4 * HL]
    ke = jnp.dot(ka, bdke_ref[...], preferred_element_type=jnp.float32)
    ko = jnp.dot(ka, bdko_ref[...], preferred_element_type=jnp.float32)
    vv = jnp.dot(va, bdv_ref[...], preferred_element_type=jnp.float32)
    cos = cos_ref[...]
    sin = sin_ref[...]
    for g in range(H):
        sl = slice(g * LAT, (g + 1) * LAT)
        b = g * 256
        pk_ref[:, b:b + LAT] = (qe[:, sl] * cos - qo[:, sl] * sin
                                ).astype(jnp.bfloat16)
        pk_ref[:, b + LAT:b + 2 * LAT] = (qe[:, sl] * sin + qo[:, sl] * cos
                                          ).astype(jnp.bfloat16)
        pk_ref[:, b + 64:b + 64 + LAT] = (
            ke[:, sl] * cos - ko[:, sl] * sin).astype(jnp.bfloat16)
        pk_ref[:, b + 96:b + 96 + LAT] = (
            ke[:, sl] * sin + ko[:, sl] * cos).astype(jnp.bfloat16)
        pk_ref[:, b + 128:b + 128 + HS] = vv[:, g * HS:(g + 1) * HS
                                             ].astype(jnp.bfloat16)


def _attn_kernel(pk_ref, o_ref):
    grp = pk_ref[...]
    qr = grp[:, :64]
    kr = grp[:, 64:128]
    v = grp[:, 128:192]

    scores = jax.lax.dot_general(qr, kr, (((1,), (1,)), ((), ())),
                                 preferred_element_type=jnp.float32)
    scores = scores / 8.0
    m = jnp.max(scores, axis=-1, keepdims=True)
    p = jnp.exp(scores - m)
    s = jnp.sum(p, axis=-1, keepdims=True)
    attn = p / s
    o_ref[0] = jnp.dot(attn, v, preferred_element_type=jnp.float32)


def _router_kernel(x_ref, o_ref, wo_ref, bo_ref, w2_ref, wr_ref, br_ref,
                   wn_ref, bn_ref, eps_ref,
                   x2_ref, h2_ref, vrank_ref, aux_ref):
    o_flat = jnp.concatenate([o_ref[hh] for hh in range(H)], axis=-1)
    x2 = x_ref[...] + jnp.dot(o_flat, wo_ref[...],
                              preferred_element_type=jnp.float32) + bo_ref[...]
    x2_ref[...] = x2
    ms = jnp.mean(x2 * x2, axis=-1, keepdims=True)
    h2 = x2 * jax.lax.rsqrt(ms + 1e-5) * w2_ref[...]
    h2_ref[...] = h2

    logits = jnp.dot(h2, wr_ref[...], preferred_element_type=jnp.float32) + br_ref[...]
    npre = jnp.dot(h2, wn_ref[...], preferred_element_type=jnp.float32) + bn_ref[...]
    nscale = jnp.logaddexp(npre, 0.0)
    noisy = logits + eps_ref[...] * nscale

    rowmax = jnp.max(noisy, axis=-1, keepdims=True)
    lane = jax.lax.broadcasted_iota(jnp.int32, (T, E), 1)
    first = jnp.min(jnp.where(noisy == rowmax, lane, E), axis=-1, keepdims=True)
    oh = (lane == first).astype(jnp.float32)

    counts = jnp.sum(oh, axis=0, keepdims=True)
    aux_ref[...] = jnp.sum((counts / T - 1.0 / E) ** 2, keepdims=True)[:, :1]

    r = jax.lax.broadcasted_iota(jnp.int32, (T, T), 0)
    cc = jax.lax.broadcasted_iota(jnp.int32, (T, T), 1)
    tri = (r >= cc).astype(jnp.bfloat16)
    rank = jax.lax.dot_general(tri, oh.astype(jnp.bfloat16),
                               (((1,), (0,)), ((), ())),
                               preferred_element_type=jnp.float32) - 1.0
    valid = (oh > 0.0) & (rank < CAP)
    vrank = jnp.where(valid, rank, -1.0)
    vrank_ref[...] = jnp.transpose(vrank)


def _expert_kernel(h2_ref, x2_ref, vrank_ref, wsw_ref, wdown_ref, out_ref):
    e = pl.program_id(0)
    vr = vrank_ref[0].astype(jnp.int32)
    slot = jax.lax.broadcasted_iota(jnp.int32, (CAP, T), 0)
    p_sel = (slot == vr).astype(jnp.float32)

    xin = jnp.dot(p_sel, h2_ref[...], preferred_element_type=jnp.float32)
    pp = jnp.dot(xin, wsw_ref[0], preferred_element_type=jnp.float32)
    p1 = pp[:, :HID]
    p2 = pp[:, HID:]
    g = (p1 * jax.lax.logistic(p1)) * p2
    eo = jnp.dot(g, wdown_ref[0], preferred_element_type=jnp.float32)
    contrib = jax.lax.dot_general(p_sel, eo, (((0,), (0,)), ((), ())),
                                  preferred_element_type=jnp.float32)

    @pl.when(e == 0)
    def _():
        out_ref[...] = x2_ref[...] + contrib

    @pl.when(e != 0)
    def _():
        out_ref[...] = out_ref[...] + contrib


def kernel(x, ln1_w, Wq, bq, Wak, Wbk, Wav, Wbv, Wo, bo, ln2_w, Wr, br, Wn,
           bn, Wsw, Wdown):
    x2d = x.reshape(T, D)

    inv_freq = 1.0 / (10000.0 ** (jnp.arange(0, HS, 2, dtype=jnp.float32) / HS))
    pos = jnp.arange(T, dtype=jnp.float32)
    ang = pos[:, None] * inv_freq[None, :]
    cos = jnp.cos(ang)
    sin = jnp.sin(ang)

    wqe = jnp.transpose(Wq[:, :, 0::2], (1, 0, 2)).reshape(D, HL)
    wqo = jnp.transpose(Wq[:, :, 1::2], (1, 0, 2)).reshape(D, HL)
    wakc = jnp.transpose(Wak, (1, 0, 2)).reshape(D, HL)
    wavc = jnp.transpose(Wav, (1, 0, 2)).reshape(D, HL)
    wcat = jnp.concatenate([wqe, wqo, wakc, wavc], axis=1)
    bcat = jnp.concatenate([bq[:, 0::2].reshape(1, HL),
                            bq[:, 1::2].reshape(1, HL),
                            jnp.zeros((1, 2 * HL), jnp.float32)], axis=1)

    eyeh = jnp.eye(H, dtype=jnp.float32)
    bdke = (eyeh[:, None, :, None] * Wbk[:, :, None, 0::2]).reshape(HL, HL)
    bdko = (eyeh[:, None, :, None] * Wbk[:, :, None, 1::2]).reshape(HL, HL)
    bdv = (eyeh[:, None, :, None] * Wbv[:, :, None, :]).reshape(HL, H * HS)

    full2 = lambda *_: (0, 0)
    pk = pl.pallas_call(
        _proj_kernel,
        in_specs=[
            pl.BlockSpec((T, D), full2),
            pl.BlockSpec((1, D), full2),
            pl.BlockSpec((D, QC), full2),
            pl.BlockSpec((1, QC), full2),
            pl.BlockSpec((HL, HL), full2),
            pl.BlockSpec((HL, HL), full2),
            pl.BlockSpec((HL, H * HS), full2),
            pl.BlockSpec((T, LAT), full2),
            pl.BlockSpec((T, LAT), full2),
        ],
        out_specs=pl.BlockSpec((T, H * 256), full2),
        out_shape=jax.ShapeDtypeStruct((T, H * 256), jnp.bfloat16),
    )(x2d, ln1_w.reshape(1, D), wcat, bcat, bdke, bdko, bdv, cos, sin)

    o = pl.pallas_call(
        _attn_kernel,
        grid=(H,),
        in_specs=[pl.BlockSpec((T, 256), lambda h: (0, h))],
        out_specs=pl.BlockSpec((1, T, HS), lambda h: (h, 0, 0)),
        out_shape=jax.ShapeDtypeStruct((H, T, HS), jnp.float32),
    )(pk)

    eps = jax.random.normal(jax.random.key(1234), (B, T, E),
                            jnp.float32).reshape(T, E)

    full = lambda *_: (0, 0)
    x2, h2, vrank_t, aux = pl.pallas_call(
        _router_kernel,
        in_specs=[
            pl.BlockSpec((T, D), full),
            pl.BlockSpec((H, T, HS), lambda: (0, 0, 0)),
            pl.BlockSpec((D, D), full),
            pl.BlockSpec((1, D), full),
            pl.BlockSpec((1, D), full),
            pl.BlockSpec((D, E), full),
            pl.BlockSpec((1, E), full),
            pl.BlockSpec((D, E), full),
            pl.BlockSpec((1, E), full),
            pl.BlockSpec((T, E), full),
        ],
        out_specs=[
            pl.BlockSpec((T, D), full),
            pl.BlockSpec((T, D), full),
            pl.BlockSpec((E, T), full),
            pl.BlockSpec((1, 1), full),
        ],
        out_shape=[
            jax.ShapeDtypeStruct((T, D), jnp.float32),
            jax.ShapeDtypeStruct((T, D), jnp.float32),
            jax.ShapeDtypeStruct((E, T), jnp.float32),
            jax.ShapeDtypeStruct((1, 1), jnp.float32),
        ],
    )(x2d, o, Wo, bo.reshape(1, D), ln2_w.reshape(1, D), Wr,
      br.reshape(1, E), Wn, bn.reshape(1, E), eps)

    vrank3 = vrank_t.reshape(E, 1, T)

    out2d = pl.pallas_call(
        _expert_kernel,
        grid=(E,),
        in_specs=[
            pl.BlockSpec((T, D), lambda e: (0, 0)),
            pl.BlockSpec((T, D), lambda e: (0, 0)),
            pl.BlockSpec((1, 1, T), lambda e: (e, 0, 0)),
            pl.BlockSpec((1, D, 2 * HID), lambda e: (e, 0, 0)),
            pl.BlockSpec((1, HID, D), lambda e: (e, 0, 0)),
        ],
        out_specs=pl.BlockSpec((T, D), lambda e: (0, 0)),
        out_shape=jax.ShapeDtypeStruct((T, D), jnp.float32),
    )(h2, x2, vrank3, Wsw, Wdown)

    return out2d.reshape(B, T, D), aux.reshape(())


# speedup vs baseline: 7.3312x; 1.1857x over previous
import jax
import jax.numpy as jnp
from jax.experimental import pallas as pl
from jax.experimental.pallas import tpu as pltpu

B, T, D = 1, 2048, 768
H, HS, LAT = 12, 64, 32
E, TOPK = 64, 1
NFF = 1536
HID = int(2 * NFF / 3)
CAP = int(B * T * TOPK / E * 1.0)

HL = H * LAT
QC = 4 * HL


def _proj_kernel(x_ref, w1_ref, wcat_ref, bcat_ref, bdke_ref, bdko_ref,
                 bdv_ref, cos_ref, sin_ref, pk_ref):
    xv = x_ref[...]
    ms = jnp.mean(xv * xv, axis=-1, keepdims=True)
    hv = xv * jax.lax.rsqrt(ms + 1e-5) * w1_ref[...]
    qav = jnp.dot(hv, wcat_ref[...],
                  preferred_element_type=jnp.float32) + bcat_ref[...]
    qe = qav[:, 0 * HL:1 * HL]
    qo = qav[:, 1 * HL:2 * HL]
    ka = qav[:, 2 * HL:3 * HL]
    va = qav[:, 3 * HL:4 * HL]
    ke = jnp.dot(ka, bdke_ref[...], preferred_element_type=jnp.float32)
    ko = jnp.dot(ka, bdko_ref[...], preferred_element_type=jnp.float32)
    vv = jnp.dot(va, bdv_ref[...], preferred_element_type=jnp.float32)
    cos = cos_ref[...]
    sin = sin_ref[...]
    for g in range(H):
        sl = slice(g * LAT, (g + 1) * LAT)
        b = g * 256
        pk_ref[:, b:b + LAT] = ((qe[:, sl] * cos - qo[:, sl] * sin) * 0.125
                                ).astype(jnp.bfloat16)
        pk_ref[:, b + LAT:b + 2 * LAT] = ((qe[:, sl] * sin + qo[:, sl] * cos)
                                          * 0.125).astype(jnp.bfloat16)
        pk_ref[:, b + 64:b + 64 + LAT] = (
            ke[:, sl] * cos - ko[:, sl] * sin).astype(jnp.bfloat16)
        pk_ref[:, b + 96:b + 96 + LAT] = (
            ke[:, sl] * sin + ko[:, sl] * cos).astype(jnp.bfloat16)
        pk_ref[:, b + 128:b + 128 + HS] = vv[:, g * HS:(g + 1) * HS
                                             ].astype(jnp.bfloat16)


def _attn_kernel(pk_ref, o_ref):
    grp = pk_ref[...]
    qr = grp[:, :64]
    kr = grp[:, 64:128]
    v = grp[:, 128:192]

    scores = jax.lax.dot_general(qr, kr, (((1,), (1,)), ((), ())),
                                 preferred_element_type=jnp.float32)
    p = jnp.exp(scores)
    s = jnp.sum(p, axis=-1, keepdims=True)
    o = jnp.dot(p, v, preferred_element_type=jnp.float32)
    o_ref[0] = o * (1.0 / s)


def _router_kernel(x_ref, o_ref, wo_ref, bo_ref, w2_ref, wr_ref, br_ref,
                   wn_ref, bn_ref, eps_ref,
                   x2_ref, h2_ref, vrank_ref, aux_ref):
    o_flat = jnp.concatenate([o_ref[hh] for hh in range(H)], axis=-1)
    x2 = x_ref[...] + jnp.dot(o_flat, wo_ref[...],
                              preferred_element_type=jnp.float32) + bo_ref[...]
    x2_ref[...] = x2
    ms = jnp.mean(x2 * x2, axis=-1, keepdims=True)
    h2 = x2 * jax.lax.rsqrt(ms + 1e-5) * w2_ref[...]
    h2_ref[...] = h2

    logits = jnp.dot(h2, wr_ref[...], preferred_element_type=jnp.float32) + br_ref[...]
    npre = jnp.dot(h2, wn_ref[...], preferred_element_type=jnp.float32) + bn_ref[...]
    nscale = jnp.logaddexp(npre, 0.0)
    noisy = logits + eps_ref[...] * nscale

    rowmax = jnp.max(noisy, axis=-1, keepdims=True)
    lane = jax.lax.broadcasted_iota(jnp.int32, (T, E), 1)
    first = jnp.min(jnp.where(noisy == rowmax, lane, E), axis=-1, keepdims=True)
    oh = (lane == first).astype(jnp.float32)

    counts = jnp.sum(oh, axis=0, keepdims=True)
    aux_ref[...] = jnp.sum((counts / T - 1.0 / E) ** 2, keepdims=True)[:, :1]

    r = jax.lax.broadcasted_iota(jnp.int32, (T, T), 0)
    cc = jax.lax.broadcasted_iota(jnp.int32, (T, T), 1)
    tri = (r >= cc).astype(jnp.bfloat16)
    rank = jax.lax.dot_general(tri, oh.astype(jnp.bfloat16),
                               (((1,), (0,)), ((), ())),
                               preferred_element_type=jnp.float32) - 1.0
    valid = (oh > 0.0) & (rank < CAP)
    vrank = jnp.where(valid, rank, -1.0)
    vrank_ref[...] = jnp.transpose(vrank)


def _expert_kernel(h2_ref, x2_ref, vrank_ref, wsw_ref, wdown_ref, out_ref):
    e = pl.program_id(0)
    vr = vrank_ref[0].astype(jnp.int32)
    slot = jax.lax.broadcasted_iota(jnp.int32, (CAP, T), 0)
    p_sel = (slot == vr).astype(jnp.float32)

    xin = jnp.dot(p_sel, h2_ref[...], preferred_element_type=jnp.float32)
    pp = jnp.dot(xin, wsw_ref[0], preferred_element_type=jnp.float32)
    p1 = pp[:, :HID]
    p2 = pp[:, HID:]
    g = (p1 * jax.lax.logistic(p1)) * p2
    eo = jnp.dot(g, wdown_ref[0], preferred_element_type=jnp.float32)
    contrib = jax.lax.dot_general(p_sel, eo, (((0,), (0,)), ((), ())),
                                  preferred_element_type=jnp.float32)

    @pl.when(e == 0)
    def _():
        out_ref[...] = x2_ref[...] + contrib

    @pl.when(e != 0)
    def _():
        out_ref[...] = out_ref[...] + contrib


def kernel(x, ln1_w, Wq, bq, Wak, Wbk, Wav, Wbv, Wo, bo, ln2_w, Wr, br, Wn,
           bn, Wsw, Wdown):
    x2d = x.reshape(T, D)

    inv_freq = 1.0 / (10000.0 ** (jnp.arange(0, HS, 2, dtype=jnp.float32) / HS))
    pos = jnp.arange(T, dtype=jnp.float32)
    ang = pos[:, None] * inv_freq[None, :]
    cos = jnp.cos(ang)
    sin = jnp.sin(ang)

    wqe = jnp.transpose(Wq[:, :, 0::2], (1, 0, 2)).reshape(D, HL)
    wqo = jnp.transpose(Wq[:, :, 1::2], (1, 0, 2)).reshape(D, HL)
    wakc = jnp.transpose(Wak, (1, 0, 2)).reshape(D, HL)
    wavc = jnp.transpose(Wav, (1, 0, 2)).reshape(D, HL)
    wcat = jnp.concatenate([wqe, wqo, wakc, wavc], axis=1)
    bcat = jnp.concatenate([bq[:, 0::2].reshape(1, HL),
                            bq[:, 1::2].reshape(1, HL),
                            jnp.zeros((1, 2 * HL), jnp.float32)], axis=1)

    eyeh = jnp.eye(H, dtype=jnp.float32)
    bdke = (eyeh[:, None, :, None] * Wbk[:, :, None, 0::2]).reshape(HL, HL)
    bdko = (eyeh[:, None, :, None] * Wbk[:, :, None, 1::2]).reshape(HL, HL)
    bdv = (eyeh[:, None, :, None] * Wbv[:, :, None, :]).reshape(HL, H * HS)

    full2 = lambda *_: (0, 0)
    pk = pl.pallas_call(
        _proj_kernel,
        in_specs=[
            pl.BlockSpec((T, D), full2),
            pl.BlockSpec((1, D), full2),
            pl.BlockSpec((D, QC), full2),
            pl.BlockSpec((1, QC), full2),
            pl.BlockSpec((HL, HL), full2),
            pl.BlockSpec((HL, HL), full2),
            pl.BlockSpec((HL, H * HS), full2),
            pl.BlockSpec((T, LAT), full2),
            pl.BlockSpec((T, LAT), full2),
        ],
        out_specs=pl.BlockSpec((T, H * 256), full2),
        out_shape=jax.ShapeDtypeStruct((T, H * 256), jnp.bfloat16),
    )(x2d, ln1_w.reshape(1, D), wcat, bcat, bdke, bdko, bdv, cos, sin)

    o = pl.pallas_call(
        _attn_kernel,
        grid=(H,),
        in_specs=[pl.BlockSpec((T, 256), lambda h: (0, h))],
        out_specs=pl.BlockSpec((1, T, HS), lambda h: (h, 0, 0)),
        out_shape=jax.ShapeDtypeStruct((H, T, HS), jnp.float32),
    )(pk)

    eps = jax.random.normal(jax.random.key(1234), (B, T, E),
                            jnp.float32).reshape(T, E)

    full = lambda *_: (0, 0)
    x2, h2, vrank_t, aux = pl.pallas_call(
        _router_kernel,
        in_specs=[
            pl.BlockSpec((T, D), full),
            pl.BlockSpec((H, T, HS), lambda: (0, 0, 0)),
            pl.BlockSpec((D, D), full),
            pl.BlockSpec((1, D), full),
            pl.BlockSpec((1, D), full),
            pl.BlockSpec((D, E), full),
            pl.BlockSpec((1, E), full),
            pl.BlockSpec((D, E), full),
            pl.BlockSpec((1, E), full),
            pl.BlockSpec((T, E), full),
        ],
        out_specs=[
            pl.BlockSpec((T, D), full),
            pl.BlockSpec((T, D), full),
            pl.BlockSpec((E, T), full),
            pl.BlockSpec((1, 1), full),
        ],
        out_shape=[
            jax.ShapeDtypeStruct((T, D), jnp.float32),
            jax.ShapeDtypeStruct((T, D), jnp.float32),
            jax.ShapeDtypeStruct((E, T), jnp.float32),
            jax.ShapeDtypeStruct((1, 1), jnp.float32),
        ],
    )(x2d, o, Wo, bo.reshape(1, D), ln2_w.reshape(1, D), Wr,
      br.reshape(1, E), Wn, bn.reshape(1, E), eps)

    vrank3 = vrank_t.reshape(E, 1, T)

    out2d = pl.pallas_call(
        _expert_kernel,
        grid=(E,),
        in_specs=[
            pl.BlockSpec((T, D), lambda e: (0, 0)),
            pl.BlockSpec((T, D), lambda e: (0, 0)),
            pl.BlockSpec((1, 1, T), lambda e: (e, 0, 0)),
            pl.BlockSpec((1, D, 2 * HID), lambda e: (e, 0, 0)),
            pl.BlockSpec((1, HID, D), lambda e: (e, 0, 0)),
        ],
        out_specs=pl.BlockSpec((T, D), lambda e: (0, 0)),
        out_shape=jax.ShapeDtypeStruct((T, D), jnp.float32),
    )(h2, x2, vrank3, Wsw, Wdown)

    return out2d.reshape(B, T, D), aux.reshape(())


# speedup vs baseline: 8.5961x; 1.1725x over previous
import jax
import jax.numpy as jnp
import numpy as np
from jax.experimental import pallas as pl

B, T, D = 1, 2048, 768
H, HS, LAT = 12, 64, 32
E, TOPK = 64, 1
NFF = 1536
HID = int(2 * NFF / 3)
CAP = int(B * T * TOPK / E * 1.0)

HL = H * LAT
QC = 4 * HL

_EPS = np.asarray(jax.random.normal(jax.random.key(1234), (B, T, E),
                                    jnp.float32)).reshape(T, E)
_INV_FREQ = 1.0 / (10000.0 ** (np.arange(0, HS, 2, dtype=np.float32) / HS))
_ANG = np.arange(T, dtype=np.float32)[:, None] * _INV_FREQ[None, :]
_COS = np.cos(_ANG).astype(np.float32)
_SIN = np.sin(_ANG).astype(np.float32)


def _proj_kernel(x_ref, w1_ref, wcat_ref, bcat_ref, bdke_ref, bdko_ref,
                 bdv_ref, cos_ref, sin_ref, pk_ref):
    xv = x_ref[...]
    ms = jnp.mean(xv * xv, axis=-1, keepdims=True)
    hv = xv * jax.lax.rsqrt(ms + 1e-5) * w1_ref[...]
    qav = jnp.dot(hv, wcat_ref[...],
                  preferred_element_type=jnp.float32) + bcat_ref[...]
    qe = qav[:, 0 * HL:1 * HL]
    qo = qav[:, 1 * HL:2 * HL]
    ka = qav[:, 2 * HL:3 * HL]
    va = qav[:, 3 * HL:4 * HL]
    ke = jnp.dot(ka, bdke_ref[...], preferred_element_type=jnp.float32)
    ko = jnp.dot(ka, bdko_ref[...], preferred_element_type=jnp.float32)
    vv = jnp.dot(va, bdv_ref[...], preferred_element_type=jnp.float32)
    cos = cos_ref[...]
    sin = sin_ref[...]
    for g in range(H):
        sl = slice(g * LAT, (g + 1) * LAT)
        b = g * 256
        pk_ref[:, b:b + LAT] = ((qe[:, sl] * cos - qo[:, sl] * sin) * 0.125
                                ).astype(jnp.bfloat16)
        pk_ref[:, b + LAT:b + 2 * LAT] = ((qe[:, sl] * sin + qo[:, sl] * cos)
                                          * 0.125).astype(jnp.bfloat16)
        pk_ref[:, b + 64:b + 64 + LAT] = (
            ke[:, sl] * cos - ko[:, sl] * sin).astype(jnp.bfloat16)
        pk_ref[:, b + 96:b + 96 + LAT] = (
            ke[:, sl] * sin + ko[:, sl] * cos).astype(jnp.bfloat16)
        pk_ref[:, b + 128:b + 128 + HS] = vv[:, g * HS:(g + 1) * HS
                                             ].astype(jnp.bfloat16)


def _attn_kernel(pk_ref, o_ref):
    grp = pk_ref[...]
    qr = grp[:, :64]
    kr = grp[:, 64:128]
    v = grp[:, 128:192]

    scores = jax.lax.dot_general(qr, kr, (((1,), (1,)), ((), ())),
                                 preferred_element_type=jnp.float32)
    p = jnp.exp(scores)
    s = jnp.sum(p, axis=-1, keepdims=True)
    o = jnp.dot(p, v, preferred_element_type=jnp.float32)
    o_ref[0] = (o * (1.0 / s)).astype(jnp.bfloat16)


def _router_kernel(x_ref, o_ref, wo_ref, bo_ref, w2_ref, wr_ref, br_ref,
                   wn_ref, bn_ref, eps_ref,
                   x2_ref, xin_ref, dest_ref, aux_ref):
    o_flat = jnp.concatenate([o_ref[hh] for hh in range(H)], axis=-1)
    x2 = x_ref[...] + jnp.dot(o_flat, wo_ref[...].astype(jnp.bfloat16),
                              preferred_element_type=jnp.float32) + bo_ref[...]
    x2_ref[...] = x2
    ms = jnp.mean(x2 * x2, axis=-1, keepdims=True)
    h2 = x2 * jax.lax.rsqrt(ms + 1e-5) * w2_ref[...]

    logits = jnp.dot(h2, wr_ref[...], preferred_element_type=jnp.float32) + br_ref[...]
    npre = jnp.dot(h2, wn_ref[...], preferred_element_type=jnp.float32) + bn_ref[...]
    nscale = jnp.logaddexp(npre, 0.0)
    noisy = logits + eps_ref[...] * nscale

    rowmax = jnp.max(noisy, axis=-1, keepdims=True)
    lane = jax.lax.broadcasted_iota(jnp.int32, (T, E), 1)
    first = jnp.min(jnp.where(noisy == rowmax, lane, E), axis=-1, keepdims=True)
    oh = (lane == first).astype(jnp.float32)

    counts = jnp.sum(oh, axis=0, keepdims=True)
    aux_ref[...] = jnp.sum((counts / T - 1.0 / E) ** 2, keepdims=True)[:, :1]

    r = jax.lax.broadcasted_iota(jnp.int32, (T, T), 0)
    cc = jax.lax.broadcasted_iota(jnp.int32, (T, T), 1)
    tri = (r >= cc).astype(jnp.bfloat16)
    rank = jax.lax.dot_general(tri, oh.astype(jnp.bfloat16),
                               (((1,), (0,)), ((), ())),
                               preferred_element_type=jnp.float32) - 1.0
    rank_sel = jnp.max(jnp.where(oh > 0.0, rank, -1.0), axis=-1, keepdims=True)
    dest = jnp.where((rank_sel >= 0.0) & (rank_sel < CAP),
                     first.astype(jnp.float32) * CAP + rank_sel,
                     -1.0)
    dest_i = dest.astype(jnp.int32)
    dest_ref[...] = jnp.transpose(dest_i)

    p_all = (r == jnp.transpose(dest_i)).astype(jnp.bfloat16)
    xin_ref[...] = jax.lax.dot_general(
        p_all, h2.astype(jnp.bfloat16), (((1,), (0,)), ((), ())),
        preferred_element_type=jnp.float32).astype(jnp.bfloat16)


def _expert_kernel(xin_ref, wsw_ref, wdown_ref, eo_ref):
    xin = xin_ref[...].astype(jnp.float32)
    pp = jnp.dot(xin, wsw_ref[0], preferred_element_type=jnp.float32)
    p1 = pp[:, :HID]
    p2 = pp[:, HID:]
    g = (p1 * jax.lax.logistic(p1)) * p2
    eo = jnp.dot(g, wdown_ref[0], preferred_element_type=jnp.float32)
    eo_ref[...] = eo.astype(jnp.bfloat16)


def _combine_kernel(x2_ref, eo_ref, dest_ref, out_ref):
    r = jax.lax.broadcasted_iota(jnp.int32, (T, T), 0)
    p_all = (r == dest_ref[...]).astype(jnp.bfloat16)
    contrib = jax.lax.dot_general(p_all, eo_ref[...],
                                  (((0,), (0,)), ((), ())),
                                  preferred_element_type=jnp.float32)
    out_ref[...] = x2_ref[...] + contrib


def kernel(x, ln1_w, Wq, bq, Wak, Wbk, Wav, Wbv, Wo, bo, ln2_w, Wr, br, Wn,
           bn, Wsw, Wdown):
    x2d = x.reshape(T, D)

    cos = jnp.asarray(_COS)
    sin = jnp.asarray(_SIN)
    eps = jnp.asarray(_EPS)

    wqe = jnp.transpose(Wq[:, :, 0::2], (1, 0, 2)).reshape(D, HL)
    wqo = jnp.transpose(Wq[:, :, 1::2], (1, 0, 2)).reshape(D, HL)
    wakc = jnp.transpose(Wak, (1, 0, 2)).reshape(D, HL)
    wavc = jnp.transpose(Wav, (1, 0, 2)).reshape(D, HL)
    wcat = jnp.concatenate([wqe, wqo, wakc, wavc], axis=1)
    bcat = jnp.concatenate([bq[:, 0::2].reshape(1, HL),
                            bq[:, 1::2].reshape(1, HL),
                            jnp.zeros((1, 2 * HL), jnp.float32)], axis=1)

    eyeh = jnp.eye(H, dtype=jnp.float32)
    bdke = (eyeh[:, None, :, None] * Wbk[:, :, None, 0::2]).reshape(HL, HL)
    bdko = (eyeh[:, None, :, None] * Wbk[:, :, None, 1::2]).reshape(HL, HL)
    bdv = (eyeh[:, None, :, None] * Wbv[:, :, None, :]).reshape(HL, H * HS)

    full2 = lambda *_: (0, 0)
    pk = pl.pallas_call(
        _proj_kernel,
        in_specs=[
            pl.BlockSpec((T, D), full2),
            pl.BlockSpec((1, D), full2),
            pl.BlockSpec((D, QC), full2),
            pl.BlockSpec((1, QC), full2),
            pl.BlockSpec((HL, HL), full2),
            pl.BlockSpec((HL, HL), full2),
            pl.BlockSpec((HL, H * HS), full2),
            pl.BlockSpec((T, LAT), full2),
            pl.BlockSpec((T, LAT), full2),
        ],
        out_specs=pl.BlockSpec((T, H * 256), full2),
        out_shape=jax.ShapeDtypeStruct((T, H * 256), jnp.bfloat16),
    )(x2d, ln1_w.reshape(1, D), wcat, bcat, bdke, bdko, bdv, cos, sin)

    o = pl.pallas_call(
        _attn_kernel,
        grid=(H,),
        in_specs=[pl.BlockSpec((T, 256), lambda h: (0, h))],
        out_specs=pl.BlockSpec((1, T, HS), lambda h: (h, 0, 0)),
        out_shape=jax.ShapeDtypeStruct((H, T, HS), jnp.bfloat16),
    )(pk)

    x2, xin_all, dest, aux = pl.pallas_call(
        _router_kernel,
        in_specs=[
            pl.BlockSpec((T, D), full2),
            pl.BlockSpec((H, T, HS), lambda: (0, 0, 0)),
            pl.BlockSpec((D, D), full2),
            pl.BlockSpec((1, D), full2),
            pl.BlockSpec((1, D), full2),
            pl.BlockSpec((D, E), full2),
            pl.BlockSpec((1, E), full2),
            pl.BlockSpec((D, E), full2),
            pl.BlockSpec((1, E), full2),
            pl.BlockSpec((T, E), full2),
        ],
        out_specs=[
            pl.BlockSpec((T, D), full2),
            pl.BlockSpec((T, D), full2),
            pl.BlockSpec((1, T), full2),
            pl.BlockSpec((1, 1), full2),
        ],
        out_shape=[
            jax.ShapeDtypeStruct((T, D), jnp.float32),
            jax.ShapeDtypeStruct((T, D), jnp.bfloat16),
            jax.ShapeDtypeStruct((1, T), jnp.int32),
            jax.ShapeDtypeStruct((1, 1), jnp.float32),
        ],
    )(x2d, o, Wo, bo.reshape(1, D), ln2_w.reshape(1, D), Wr,
      br.reshape(1, E), Wn, bn.reshape(1, E), eps)

    eo_all = pl.pallas_call(
        _expert_kernel,
        grid=(E,),
        in_specs=[
            pl.BlockSpec((CAP, D), lambda e: (e, 0)),
            pl.BlockSpec((1, D, 2 * HID), lambda e: (e, 0, 0)),
            pl.BlockSpec((1, HID, D), lambda e: (e, 0, 0)),
        ],
        out_specs=pl.BlockSpec((CAP, D), lambda e: (e, 0)),
        out_shape=jax.ShapeDtypeStruct((T, D), jnp.bfloat16),
    )(xin_all, Wsw, Wdown)

    out2d = pl.pallas_call(
        _combine_kernel,
        in_specs=[
            pl.BlockSpec((T, D), full2),
            pl.BlockSpec((T, D), full2),
            pl.BlockSpec((1, T), full2),
        ],
        out_specs=pl.BlockSpec((T, D), full2),
        out_shape=jax.ShapeDtypeStruct((T, D), jnp.float32),
    )(x2, eo_all, dest)

    return out2d.reshape(B, T, D), aux.reshape(())


# speedup vs baseline: 8.6026x; 1.0008x over previous
import jax
import jax.numpy as jnp
import numpy as np
from jax.experimental import pallas as pl
from jax.experimental.pallas import tpu as pltpu

B, T, D = 1, 2048, 768
H, HS, LAT = 12, 64, 32
E, TOPK = 64, 1
NFF = 1536
HID = int(2 * NFF / 3)
CAP = int(B * T * TOPK / E * 1.0)

HL = H * LAT
QC = 4 * HL

_EPS = np.asarray(jax.random.normal(jax.random.key(1234), (B, T, E),
                                    jnp.float32)).reshape(T, E)
_INV_FREQ = 1.0 / (10000.0 ** (np.arange(0, HS, 2, dtype=np.float32) / HS))
_ANG = np.arange(T, dtype=np.float32)[:, None] * _INV_FREQ[None, :]
_COS = np.cos(_ANG).astype(np.float32)
_SIN = np.sin(_ANG).astype(np.float32)


def _proj_kernel(x_ref, w1_ref, wcat_ref, bcat_ref, bdke_ref, bdko_ref,
                 bdv_ref, cos_ref, sin_ref, pk_ref):
    xv = x_ref[...]
    ms = jnp.mean(xv * xv, axis=-1, keepdims=True)
    hv = xv * jax.lax.rsqrt(ms + 1e-5) * w1_ref[...]
    qav = jnp.dot(hv, wcat_ref[...],
                  preferred_element_type=jnp.float32) + bcat_ref[...]
    qe = qav[:, 0 * HL:1 * HL]
    qo = qav[:, 1 * HL:2 * HL]
    ka = qav[:, 2 * HL:3 * HL]
    va = qav[:, 3 * HL:4 * HL]
    ke = jnp.dot(ka, bdke_ref[...], preferred_element_type=jnp.float32)
    ko = jnp.dot(ka, bdko_ref[...], preferred_element_type=jnp.float32)
    vv = jnp.dot(va, bdv_ref[...], preferred_element_type=jnp.float32)
    cos = cos_ref[...]
    sin = sin_ref[...]
    for g in range(H):
        sl = slice(g * LAT, (g + 1) * LAT)
        b = g * 256
        pk_ref[:, b:b + LAT] = ((qe[:, sl] * cos - qo[:, sl] * sin) * 0.125
                                ).astype(jnp.bfloat16)
        pk_ref[:, b + LAT:b + 2 * LAT] = ((qe[:, sl] * sin + qo[:, sl] * cos)
                                          * 0.125).astype(jnp.bfloat16)
        pk_ref[:, b + 64:b + 64 + LAT] = (
            ke[:, sl] * cos - ko[:, sl] * sin).astype(jnp.bfloat16)
        pk_ref[:, b + 96:b + 96 + LAT] = (
            ke[:, sl] * sin + ko[:, sl] * cos).astype(jnp.bfloat16)
        pk_ref[:, b + 128:b + 128 + HS] = vv[:, g * HS:(g + 1) * HS
                                             ].astype(jnp.bfloat16)


def _attn_kernel(pk_ref, o_ref):
    grp = pk_ref[...]
    qr = grp[:, :64]
    kr = grp[:, 64:128]
    v = grp[:, 128:192]

    scores = jax.lax.dot_general(qr, kr, (((1,), (1,)), ((), ())),
                                 preferred_element_type=jnp.float32)
    p = jnp.exp(scores)
    s = jnp.sum(p, axis=-1, keepdims=True)
    o = jnp.dot(p, v, preferred_element_type=jnp.float32)
    o_ref[0] = (o * (1.0 / s)).astype(jnp.bfloat16)


def _router_kernel(x_ref, o_ref, wo_ref, bo_ref, w2_ref, wr_ref, br_ref,
                   wn_ref, bn_ref, eps_ref,
                   x2_ref, xin_ref, dest_ref, aux_ref):
    o_flat = jnp.concatenate([o_ref[hh] for hh in range(H)], axis=-1)
    x2 = x_ref[...] + jnp.dot(o_flat, wo_ref[...].astype(jnp.bfloat16),
                              preferred_element_type=jnp.float32) + bo_ref[...]
    x2_ref[...] = x2
    ms = jnp.mean(x2 * x2, axis=-1, keepdims=True)
    h2 = x2 * jax.lax.rsqrt(ms + 1e-5) * w2_ref[...]

    logits = jnp.dot(h2, wr_ref[...], preferred_element_type=jnp.float32) + br_ref[...]
    npre = jnp.dot(h2, wn_ref[...], preferred_element_type=jnp.float32) + bn_ref[...]
    nscale = jnp.logaddexp(npre, 0.0)
    noisy = logits + eps_ref[...] * nscale

    rowmax = jnp.max(noisy, axis=-1, keepdims=True)
    lane = jax.lax.broadcasted_iota(jnp.int32, (T, E), 1)
    first = jnp.min(jnp.where(noisy == rowmax, lane, E), axis=-1, keepdims=True)
    oh = (lane == first).astype(jnp.float32)

    counts = jnp.sum(oh, axis=0, keepdims=True)
    aux_ref[...] = jnp.sum((counts / T - 1.0 / E) ** 2, keepdims=True)[:, :1]

    r = jax.lax.broadcasted_iota(jnp.int32, (T, T), 0)
    cc = jax.lax.broadcasted_iota(jnp.int32, (T, T), 1)
    tri = (r >= cc).astype(jnp.bfloat16)
    rank = jax.lax.dot_general(tri, oh.astype(jnp.bfloat16),
                               (((1,), (0,)), ((), ())),
                               preferred_element_type=jnp.float32) - 1.0
    rank_sel = jnp.max(jnp.where(oh > 0.0, rank, -1.0), axis=-1, keepdims=True)
    dest = jnp.where((rank_sel >= 0.0) & (rank_sel < CAP),
                     first.astype(jnp.float32) * CAP + rank_sel,
                     -1.0)
    dest_i = dest.astype(jnp.int32)
    dest_ref[...] = jnp.transpose(dest_i)

    p_all = (r == jnp.transpose(dest_i)).astype(jnp.bfloat16)
    xin_ref[...] = jax.lax.dot_general(
        p_all, h2.astype(jnp.bfloat16), (((1,), (0,)), ((), ())),
        preferred_element_type=jnp.float32).astype(jnp.bfloat16)


def _expert_kernel(xin_ref, wsw_ref, wdown_ref, eo_ref):
    xin = xin_ref[...].astype(jnp.float32)
    pp = jnp.dot(xin, wsw_ref[0], preferred_element_type=jnp.float32)
    p1 = pp[:, :HID]
    p2 = pp[:, HID:]
    g = (p1 * jax.lax.logistic(p1)) * p2
    eo = jnp.dot(g, wdown_ref[0], preferred_element_type=jnp.float32)
    eo_ref[...] = eo.astype(jnp.bfloat16)


def _combine_kernel(x2_ref, eo_ref, dest_ref, out_ref):
    r = jax.lax.broadcasted_iota(jnp.int32, (T, T), 0)
    p_all = (r == dest_ref[...]).astype(jnp.bfloat16)
    contrib = jax.lax.dot_general(p_all, eo_ref[...],
                                  (((0,), (0,)), ((), ())),
                                  preferred_element_type=jnp.float32)
    out_ref[...] = x2_ref[...] + contrib


def kernel(x, ln1_w, Wq, bq, Wak, Wbk, Wav, Wbv, Wo, bo, ln2_w, Wr, br, Wn,
           bn, Wsw, Wdown):
    x2d = x.reshape(T, D)

    cos = jnp.asarray(_COS)
    sin = jnp.asarray(_SIN)
    eps = jnp.asarray(_EPS)

    wqe = jnp.transpose(Wq[:, :, 0::2], (1, 0, 2)).reshape(D, HL)
    wqo = jnp.transpose(Wq[:, :, 1::2], (1, 0, 2)).reshape(D, HL)
    wakc = jnp.transpose(Wak, (1, 0, 2)).reshape(D, HL)
    wavc = jnp.transpose(Wav, (1, 0, 2)).reshape(D, HL)
    wcat = jnp.concatenate([wqe, wqo, wakc, wavc], axis=1)
    bcat = jnp.concatenate([bq[:, 0::2].reshape(1, HL),
                            bq[:, 1::2].reshape(1, HL),
                            jnp.zeros((1, 2 * HL), jnp.float32)], axis=1)

    eyeh = jnp.eye(H, dtype=jnp.float32)
    bdke = (eyeh[:, None, :, None] * Wbk[:, :, None, 0::2]).reshape(HL, HL)
    bdko = (eyeh[:, None, :, None] * Wbk[:, :, None, 1::2]).reshape(HL, HL)
    bdv = (eyeh[:, None, :, None] * Wbv[:, :, None, :]).reshape(HL, H * HS)

    full2 = lambda *_: (0, 0)
    pk = pl.pallas_call(
        _proj_kernel,
        in_specs=[
            pl.BlockSpec((T, D), full2),
            pl.BlockSpec((1, D), full2),
            pl.BlockSpec((D, QC), full2),
            pl.BlockSpec((1, QC), full2),
            pl.BlockSpec((HL, HL), full2),
            pl.BlockSpec((HL, HL), full2),
            pl.BlockSpec((HL, H * HS), full2),
            pl.BlockSpec((T, LAT), full2),
            pl.BlockSpec((T, LAT), full2),
        ],
        out_specs=pl.BlockSpec((T, H * 256), full2),
        out_shape=jax.ShapeDtypeStruct((T, H * 256), jnp.bfloat16),
    )(x2d, ln1_w.reshape(1, D), wcat, bcat, bdke, bdko, bdv, cos, sin)

    o = pl.pallas_call(
        _attn_kernel,
        grid=(H,),
        in_specs=[pl.BlockSpec((T, 256), lambda h: (0, h))],
        out_specs=pl.BlockSpec((1, T, HS), lambda h: (h, 0, 0)),
        out_shape=jax.ShapeDtypeStruct((H, T, HS), jnp.bfloat16),
        compiler_params=pltpu.CompilerParams(
            dimension_semantics=("parallel",)),
    )(pk)

    x2, xin_all, dest, aux = pl.pallas_call(
        _router_kernel,
        in_specs=[
            pl.BlockSpec((T, D), full2),
            pl.BlockSpec((H, T, HS), lambda: (0, 0, 0)),
            pl.BlockSpec((D, D), full2),
            pl.BlockSpec((1, D), full2),
            pl.BlockSpec((1, D), full2),
            pl.BlockSpec((D, E), full2),
            pl.BlockSpec((1, E), full2),
            pl.BlockSpec((D, E), full2),
            pl.BlockSpec((1, E), full2),
            pl.BlockSpec((T, E), full2),
        ],
        out_specs=[
            pl.BlockSpec((T, D), full2),
            pl.BlockSpec((T, D), full2),
            pl.BlockSpec((1, T), full2),
            pl.BlockSpec((1, 1), full2),
        ],
        out_shape=[
            jax.ShapeDtypeStruct((T, D), jnp.float32),
            jax.ShapeDtypeStruct((T, D), jnp.bfloat16),
            jax.ShapeDtypeStruct((1, T), jnp.int32),
            jax.ShapeDtypeStruct((1, 1), jnp.float32),
        ],
    )(x2d, o, Wo, bo.reshape(1, D), ln2_w.reshape(1, D), Wr,
      br.reshape(1, E), Wn, bn.reshape(1, E), eps)

    eo_all = pl.pallas_call(
        _expert_kernel,
        grid=(E,),
        in_specs=[
            pl.BlockSpec((CAP, D), lambda e: (e, 0)),
            pl.BlockSpec((1, D, 2 * HID), lambda e: (e, 0, 0)),
            pl.BlockSpec((1, HID, D), lambda e: (e, 0, 0)),
        ],
        out_specs=pl.BlockSpec((CAP, D), lambda e: (e, 0)),
        out_shape=jax.ShapeDtypeStruct((T, D), jnp.bfloat16),
        compiler_params=pltpu.CompilerParams(
            dimension_semantics=("parallel",)),
    )(xin_all, Wsw, Wdown)

    out2d = pl.pallas_call(
        _combine_kernel,
        in_specs=[
            pl.BlockSpec((T, D), full2),
            pl.BlockSpec((T, D), full2),
            pl.BlockSpec((1, T), full2),
        ],
        out_specs=pl.BlockSpec((T, D), full2),
        out_shape=jax.ShapeDtypeStruct((T, D), jnp.float32),
    )(x2, eo_all, dest)

    return out2d.reshape(B, T, D), aux.reshape(())
